# Initial kernel scaffold; baseline (speedup 1.0000x reference)
import jax
import jax.numpy as jnp
from jax.experimental import pallas as pl


def kernel(x, edge_index, batch, W, b):
    raise NotImplementedError("write your pallas kernel here")


# speedup vs baseline: 16.9398x; 16.9398x over previous
import functools

import jax
import jax.numpy as jnp
from jax import lax
from jax.experimental import pallas as pl
from jax.experimental.pallas import tpu as pltpu
from jax.experimental.pallas import tpu_sc as plsc

N = 10000
E = 320000
D = 128
NPAD = 10240
NC, NS = 2, 16
NW = NC * NS
C = 128
CHUNKS = 80
EPT = CHUNKS * C
IG = 8
NG = CHUNKS // IG
ROWS_PER_SUB = NPAD // NS

_sc_mesh = plsc.VectorSubcoreMesh(core_axis_name="c", subcore_axis_name="s")


@functools.partial(
    pl.kernel,
    out_type=jax.ShapeDtypeStruct((NC * NPAD,), jnp.float32),
    mesh=_sc_mesh,
    scratch_types=[
        pltpu.VMEM((CHUNKS, C), jnp.int32),
        pltpu.VMEM((C,), jnp.float32),
        pltpu.VMEM((ROWS_PER_SUB,), jnp.float32),
        pltpu.VMEM_SHARED((NPAD,), jnp.float32),
    ],
)
def _sc_histogram(dst_hbm, hist_hbm, idx_v, ones_v, zero_v, hist_sp):
    c = lax.axis_index("c")
    s = lax.axis_index("s")
    wid = c * NS + s

    @pl.loop(0, C, step=16)
    def _(i):
        ones_v[pl.ds(i, 16)] = jnp.ones((16,), jnp.float32)

    @pl.loop(0, ROWS_PER_SUB, step=16)
    def _(i):
        zero_v[pl.ds(i, 16)] = jnp.zeros((16,), jnp.float32)

    pltpu.sync_copy(zero_v, hist_sp.at[pl.ds(s * ROWS_PER_SUB, ROWS_PER_SUB)])
    plsc.subcore_barrier()

    pltpu.sync_copy(dst_hbm.at[pl.ds(wid * CHUNKS, CHUNKS)], idx_v)

    @pl.loop(0, CHUNKS)
    def _(j):
        pltpu.sync_copy(ones_v, hist_sp.at[idx_v.at[j]], add=True)

    plsc.subcore_barrier()
    pltpu.sync_copy(
        hist_sp.at[pl.ds(s * ROWS_PER_SUB, ROWS_PER_SUB)],
        hist_hbm.at[pl.ds(c * NPAD + s * ROWS_PER_SUB, ROWS_PER_SUB)],
    )


@functools.partial(
    pl.kernel,
    out_type=jax.ShapeDtypeStruct((NC, NPAD, D), jnp.float32),
    mesh=_sc_mesh,
    scratch_types=[
        pltpu.VMEM((2, IG, C), jnp.int32),
        pltpu.VMEM((2, IG, C), jnp.int32),
        pltpu.VMEM((C, D), jnp.float32),
        pltpu.VMEM((C, D), jnp.float32),
        pltpu.VMEM_SHARED((NPAD, D), jnp.float32),
        pltpu.SemaphoreType.DMA,
        pltpu.SemaphoreType.DMA,
        pltpu.SemaphoreType.DMA,
    ],
)
def _sc_propagate(hs_hbm, src_hbm, dst_hbm, acc_hbm,
                  src_v, dst_v, buf0, buf1, acc_sp, sem0, sem1, isem):
    c = lax.axis_index("c")
    s = lax.axis_index("s")
    wid = c * NS + s
    base = wid * CHUNKS

    @pl.loop(0, C)
    def _(r):
        @pl.loop(0, D, step=16)
        def _(cc):
            buf0[r, pl.ds(cc, 16)] = jnp.zeros((16,), jnp.float32)

    @pl.loop(0, ROWS_PER_SUB, step=C)
    def _(r):
        pltpu.sync_copy(buf0, acc_sp.at[pl.ds(s * ROWS_PER_SUB + r, C)])

    plsc.subcore_barrier()

    pltpu.sync_copy(src_hbm.at[pl.ds(base, IG)], src_v.at[0])
    pltpu.sync_copy(dst_hbm.at[pl.ds(base, IG)], dst_v.at[0])
    pltpu.async_copy(hs_hbm.at[src_v.at[0, 0]], buf0, sem0)

    @pl.loop(0, NG)
    def _(g):
        p = lax.rem(g, 2)
        pn = 1 - p
        nxt = base + (g + 1) * IG

        @pl.when(g + 1 < NG)
        def _():
            pltpu.async_copy(src_hbm.at[pl.ds(nxt, IG)], src_v.at[pn], isem)
            pltpu.async_copy(dst_hbm.at[pl.ds(nxt, IG)], dst_v.at[pn], isem)

        @pl.loop(0, IG, step=2)
        def _(j):
            pltpu.async_copy(hs_hbm.at[src_v.at[p, j + 1]], buf1, sem1)
            pltpu.make_async_copy(hs_hbm.at[src_v.at[p, j]], buf0, sem0).wait()
            pltpu.sync_copy(buf0, acc_sp.at[dst_v.at[p, j]], add=True)

            @pl.when(j + 2 < IG)
            def _():
                pltpu.async_copy(hs_hbm.at[src_v.at[p, j + 2]], buf0, sem0)

            @pl.when(jnp.logical_and(j + 2 >= IG, g + 1 < NG))
            def _():
                pltpu.make_async_copy(
                    src_hbm.at[pl.ds(nxt, IG)], src_v.at[pn], isem).wait()
                pltpu.make_async_copy(
                    dst_hbm.at[pl.ds(nxt, IG)], dst_v.at[pn], isem).wait()
                pltpu.async_copy(hs_hbm.at[src_v.at[pn, 0]], buf0, sem0)

            pltpu.make_async_copy(
                hs_hbm.at[src_v.at[p, j + 1]], buf1, sem1).wait()
            pltpu.sync_copy(buf1, acc_sp.at[dst_v.at[p, j + 1]], add=True)

    plsc.subcore_barrier()
    pltpu.sync_copy(
        acc_sp.at[pl.ds(s * ROWS_PER_SUB, ROWS_PER_SUB)],
        acc_hbm.at[c].at[pl.ds(s * ROWS_PER_SUB, ROWS_PER_SUB)],
    )


def _mm_body(x_ref, w_ref, o_ref):
    o_ref[...] = jnp.dot(x_ref[...], w_ref[...],
                         preferred_element_type=jnp.float32)


def _scale_body(hist_ref, h_ref, o_ref):
    deg = hist_ref[0] + hist_ref[1] + 2.0
    dinv = lax.rsqrt(deg)
    hb = h_ref[...].reshape(8, 128, D)
    o_ref[...] = (hb * dinv[:, :, None]).reshape(1024, D)


def _final_body(hist_ref, acc_ref, hs_ref, b_ref, o_ref):
    deg = hist_ref[0] + hist_ref[1] + 2.0
    dinv = lax.rsqrt(deg)
    t = acc_ref[0] + acc_ref[1] + 2.0 * hs_ref[...]
    t = (t.reshape(8, 128, D) * dinv[:, :, None]).reshape(1024, D)
    t = jnp.maximum(t + b_ref[...], 0.0)
    m = jnp.max(t, axis=-1, keepdims=True)
    lse = jnp.log(jnp.sum(jnp.exp(t - m), axis=-1, keepdims=True)) + m
    o_ref[...] = t - lse


_GRID = NPAD // 1024
_row_spec = pl.BlockSpec((1024, D), lambda i: (i, 0))
_hist_spec = pl.BlockSpec((2, 8, D), lambda i: (0, i, 0))


@jax.jit
def kernel(x, edge_index, batch, W, b):
    pad = jnp.full((NW, EPT - E // NW), NPAD - 1, jnp.int32)
    src2d = jnp.concatenate(
        [edge_index[0].reshape(NW, E // NW), pad], axis=1).reshape(-1, C)
    dst2d = jnp.concatenate(
        [edge_index[1].reshape(NW, E // NW), pad], axis=1).reshape(-1, C)
    xpad = jnp.pad(x, ((0, NPAD - N), (0, 0)))

    hist = _sc_histogram(dst2d)
    hist3 = hist.reshape(NC, NPAD // D, D)

    h = pl.pallas_call(
        _mm_body,
        grid=(_GRID,),
        in_specs=[_row_spec, pl.BlockSpec((D, D), lambda i: (0, 0))],
        out_specs=_row_spec,
        out_shape=jax.ShapeDtypeStruct((NPAD, D), jnp.float32),
    )(xpad, W)

    hs = pl.pallas_call(
        _scale_body,
        grid=(_GRID,),
        in_specs=[_hist_spec, _row_spec],
        out_specs=_row_spec,
        out_shape=jax.ShapeDtypeStruct((NPAD, D), jnp.float32),
    )(hist3, h)

    acc = _sc_propagate(hs, src2d, dst2d)

    out = pl.pallas_call(
        _final_body,
        grid=(_GRID,),
        in_specs=[_hist_spec,
                  pl.BlockSpec((2, 1024, D), lambda i: (0, i, 0)),
                  _row_spec,
                  pl.BlockSpec((1, D), lambda i: (0, 0))],
        out_specs=_row_spec,
        out_shape=jax.ShapeDtypeStruct((NPAD, D), jnp.float32),
    )(hist3, acc, hs, b.reshape(1, D))

    return out[:N]
